# baseline (speedup 1.0000x reference)
import functools

import jax
import jax.numpy as jnp
from jax import lax
from jax.experimental import pallas as pl
from jax.experimental.pallas import tpu as pltpu
from jax.experimental.pallas import tpu_sc as plsc

B = 4096
L = 50
LP = 56
D = 64
W = 1000000
NC = 2
NS = 16
NW = NC * NS
BPW = B // NW
NCHUNK = D // 16
NBUF = 4
SLOTS = 2 * LP


def _pool_kernel(d_h, m_h, tab_h, out_h,
                 idx_v, par_v, m_v, g0, g1, g2, g3, out_v,
                 s0, s1, s2, s3):
    wid = lax.axis_index("s") * NC + lax.axis_index("c")
    base = wid * BPW
    gs = (g0, g1, g2, g3)
    ss = (s0, s1, s2, s3)

    pltpu.sync_copy(d_h.at[pl.ds(base, BPW)], idx_v)
    pltpu.sync_copy(m_h.at[pl.ds(base, BPW)], m_v)

    def prep(b, carry):
        for c in range(SLOTS // 16):
            v = idx_v[b, pl.ds(c * 16, 16)]
            idx_v[b, pl.ds(c * 16, 16)] = lax.shift_right_logical(v, 1)
            par_v[b, pl.ds(c * 16, 16)] = lax.bitwise_and(v, 1)
        return carry

    lax.fori_loop(0, BPW, prep, 0)

    def issue(b, g, s):
        pltpu.async_copy(tab_h.at[idx_v.at[b, pl.ds(0, SLOTS)]], g, s)

    def wait(b, g, s):
        pltpu.make_async_copy(tab_h.at[idx_v.at[b, pl.ds(0, SLOTS)]], g, s).wait()

    def compute(b, g):
        mrow = [m_v[b, pl.ds(c * 16, 16)] for c in range(2 * D // 16)]
        prow = [par_v[b, pl.ds(c * 16, 16)] for c in range(SLOTS // 16)]
        z = jnp.zeros((16,), jnp.float32)
        accs1 = [z] * NCHUNK
        accs2 = [z] * NCHUNK
        for l in range(L):
            c, j = divmod(l, 16)
            c2, j2 = divmod(LP + l, 16)
            mv1 = jnp.full((16,), mrow[c][j], jnp.float32)
            mv2 = jnp.full((16,), mrow[NCHUNK + c][j], jnp.float32)
            o1 = prow[c][j] * D
            o2 = prow[c2][j2] * D
            for k in range(NCHUNK):
                accs1[k] = accs1[k] + g[l, pl.ds(o1 + k * 16, 16)] * mv1
                accs2[k] = accs2[k] + g[LP + l, pl.ds(o2 + k * 16, 16)] * mv2
        for k in range(NCHUNK):
            out_v[b, pl.ds(k * 16, 16)] = accs1[k]
        for k in range(NCHUNK):
            out_v[b, pl.ds(D + k * 16, 16)] = accs2[k]

    for j in range(NBUF):
        issue(j, gs[j], ss[j])

    def step(t, carry):
        for j in range(NBUF):
            b = t * NBUF + j
            wait(b, gs[j], ss[j])
            compute(b, gs[j])

            @pl.when(b + NBUF < BPW)
            def _():
                issue(b + NBUF, gs[j], ss[j])

        return carry

    lax.fori_loop(0, BPW // NBUF, step, 0)

    pltpu.sync_copy(out_v, out_h.at[pl.ds(base, BPW)])


def _pool(d, m, table2):
    mesh = plsc.VectorSubcoreMesh(core_axis_name="c", subcore_axis_name="s")
    f = functools.partial(
        pl.kernel,
        out_type=jax.ShapeDtypeStruct((B, 2 * D), jnp.float32),
        mesh=mesh,
        scratch_types=[
            pltpu.VMEM((BPW, SLOTS), jnp.int32),
            pltpu.VMEM((BPW, SLOTS), jnp.int32),
            pltpu.VMEM((BPW, 2 * D), jnp.float32),
            pltpu.VMEM((SLOTS, 2 * D), jnp.float32),
            pltpu.VMEM((SLOTS, 2 * D), jnp.float32),
            pltpu.VMEM((SLOTS, 2 * D), jnp.float32),
            pltpu.VMEM((SLOTS, 2 * D), jnp.float32),
            pltpu.VMEM((BPW, 2 * D), jnp.float32),
            pltpu.SemaphoreType.DMA,
            pltpu.SemaphoreType.DMA,
            pltpu.SemaphoreType.DMA,
            pltpu.SemaphoreType.DMA,
        ],
        compiler_params=pltpu.CompilerParams(use_tc_tiling_on_sc=False),
    )(_pool_kernel)
    return f(d, m, table2)


def _mlp_body(opt_ref, w1_ref, b1_ref, w2_ref, b2_ref, o_ref):
    opt = opt_ref[...] * (1.0 / L)
    h = jnp.dot(opt, w1_ref[...], preferred_element_type=jnp.float32)
    h = h + b1_ref[...]
    o = jnp.dot(h, w2_ref[...], preferred_element_type=jnp.float32)
    o_ref[...] = o + b2_ref[...]


def kernel(data_x1, mask_x1, data_x2, mask_x2, table, W1, b1, W2, b2):
    iota_b = jnp.arange(B, dtype=jnp.int32)[:, None]
    iota_j = jnp.arange(LP - L, dtype=jnp.int32)[None, :]
    pad1 = (iota_b * 2 * (LP - L) + iota_j) % W
    pad2 = (iota_b * 2 * (LP - L) + (LP - L) + iota_j) % W
    d1 = jnp.concatenate((data_x1.astype(jnp.int32), pad1), axis=1)
    d2 = jnp.concatenate((data_x2.astype(jnp.int32), pad2), axis=1)
    d = jnp.concatenate((d1, d2), axis=1)
    m1 = jnp.pad(mask_x1, ((0, 0), (0, D - L)))
    m2 = jnp.pad(mask_x2, ((0, 0), (0, D - L)))
    m = jnp.concatenate((m1, m2), axis=1)
    table2 = table.reshape(W // 2, 2 * D)
    opt = _pool(d, m, table2)
    return pl.pallas_call(
        _mlp_body,
        out_shape=jax.ShapeDtypeStruct((B, 2), jnp.float32),
    )(opt, W1, b1.reshape(1, -1), W2, b2.reshape(1, -1))


# speedup vs baseline: 1.0993x; 1.0993x over previous
import functools

import jax
import jax.numpy as jnp
from jax import lax
from jax.experimental import pallas as pl
from jax.experimental.pallas import tpu as pltpu
from jax.experimental.pallas import tpu_sc as plsc

B = 4096
L = 50
LP = 56
D = 64
W = 1000000
NC = 2
NS = 16
NW = NC * NS
BPW = B // NW
NCHUNK = D // 16
NBUF = 4
SLOTS = 2 * LP


def _pool_kernel(d_h, m_h, tab_h, out_h,
                 idx_v, m_v, g0, g1, g2, g3, out_v,
                 s0, s1, s2, s3):
    wid = lax.axis_index("s") * NC + lax.axis_index("c")
    base = wid * BPW
    gs = (g0, g1, g2, g3)
    ss = (s0, s1, s2, s3)

    pltpu.sync_copy(d_h.at[pl.ds(base, BPW)], idx_v)
    pltpu.sync_copy(m_h.at[pl.ds(base, BPW)], m_v)

    def issue(b, g, s):
        pltpu.async_copy(tab_h.at[idx_v.at[b, pl.ds(0, SLOTS)]], g, s)

    def wait(b, g, s):
        pltpu.make_async_copy(tab_h.at[idx_v.at[b, pl.ds(0, SLOTS)]], g, s).wait()

    def compute(b, g):
        mrow = [m_v[b, pl.ds(c * 16, 16)] for c in range(2 * D // 16)]
        z = jnp.zeros((16,), jnp.float32)
        accs1 = [z] * NCHUNK
        accs2 = [z] * NCHUNK
        for l in range(L):
            c, j = divmod(l, 16)
            mv1 = jnp.full((16,), mrow[c][j], jnp.float32)
            mv2 = jnp.full((16,), mrow[NCHUNK + c][j], jnp.float32)
            for k in range(NCHUNK):
                accs1[k] = accs1[k] + g[l, pl.ds(k * 16, 16)] * mv1
                accs2[k] = accs2[k] + g[LP + l, pl.ds(k * 16, 16)] * mv2
        for k in range(NCHUNK):
            out_v[b, pl.ds(k * 16, 16)] = accs1[k]
        for k in range(NCHUNK):
            out_v[b, pl.ds(D + k * 16, 16)] = accs2[k]

    for j in range(NBUF):
        issue(j, gs[j], ss[j])

    def step(t, carry):
        for j in range(NBUF):
            b = t * NBUF + j
            wait(b, gs[j], ss[j])
            compute(b, gs[j])

            @pl.when(b + NBUF < BPW)
            def _():
                issue(b + NBUF, gs[j], ss[j])

        return carry

    lax.fori_loop(0, BPW // NBUF, step, 0)

    pltpu.sync_copy(out_v, out_h.at[pl.ds(base, BPW)])


def _pool(d, m, table2):
    mesh = plsc.VectorSubcoreMesh(core_axis_name="c", subcore_axis_name="s")
    f = functools.partial(
        pl.kernel,
        out_type=jax.ShapeDtypeStruct((B, 2 * D), jnp.float32),
        mesh=mesh,
        scratch_types=[
            pltpu.VMEM((BPW, SLOTS), jnp.int32),
            pltpu.VMEM((BPW, 2 * D), jnp.float32),
            pltpu.VMEM((SLOTS, 2 * D), jnp.float32),
            pltpu.VMEM((SLOTS, 2 * D), jnp.float32),
            pltpu.VMEM((SLOTS, 2 * D), jnp.float32),
            pltpu.VMEM((SLOTS, 2 * D), jnp.float32),
            pltpu.VMEM((BPW, 2 * D), jnp.float32),
            pltpu.SemaphoreType.DMA,
            pltpu.SemaphoreType.DMA,
            pltpu.SemaphoreType.DMA,
            pltpu.SemaphoreType.DMA,
        ],
        compiler_params=pltpu.CompilerParams(use_tc_tiling_on_sc=False),
    )(_pool_kernel)
    return f(d, m, table2)


def _mlp_body(opt_ref, w1_ref, b1_ref, w2_ref, b2_ref, o_ref):
    opt = opt_ref[...] * (1.0 / L)
    h = jnp.dot(opt, w1_ref[...], preferred_element_type=jnp.float32)
    h = h + b1_ref[...]
    o = jnp.dot(h, w2_ref[...], preferred_element_type=jnp.float32)
    o_ref[...] = o + b2_ref[...]


def kernel(data_x1, mask_x1, data_x2, mask_x2, table, W1, b1, W2, b2):
    iota_b = jnp.arange(B, dtype=jnp.int32)[:, None]
    iota_j = jnp.arange(LP - L, dtype=jnp.int32)[None, :]
    pad1 = (iota_b * 2 * (LP - L) + iota_j) % W
    pad2 = (iota_b * 2 * (LP - L) + (LP - L) + iota_j) % W
    d1 = jnp.concatenate((data_x1.astype(jnp.int32), pad1), axis=1)
    d2 = jnp.concatenate((data_x2.astype(jnp.int32), pad2), axis=1)
    d = jnp.concatenate((d1, d2), axis=1)
    m1 = jnp.pad(mask_x1, ((0, 0), (0, D - L)))
    m2 = jnp.pad(mask_x2, ((0, 0), (0, D - L)))
    m = jnp.concatenate((m1, m2), axis=1)
    table2 = jnp.pad(table, ((0, 0), (0, D)))
    opt = _pool(d, m, table2)
    return pl.pallas_call(
        _mlp_body,
        out_shape=jax.ShapeDtypeStruct((B, 2), jnp.float32),
    )(opt, W1, b1.reshape(1, -1), W2, b2.reshape(1, -1))


# speedup vs baseline: 1.1548x; 1.0505x over previous
import functools

import jax
import jax.numpy as jnp
from jax import lax
from jax.experimental import pallas as pl
from jax.experimental.pallas import tpu as pltpu
from jax.experimental.pallas import tpu_sc as plsc

B = 4096
L = 50
LP = 56
D = 64
W = 1000000
NC = 2
NS = 16
NW = NC * NS
BPW = B // NW
NCHUNK = D // 16
NBUF = 4
SLOTS = 2 * LP


def _pool_kernel(d_h, m_h, tab_h, out_h,
                 idx_v, m_v, g0, g1, g2, g3, out_v,
                 s0, s1, s2, s3):
    wid = lax.axis_index("s") * NC + lax.axis_index("c")
    base = wid * BPW
    gs = (g0, g1, g2, g3)
    ss = (s0, s1, s2, s3)

    pltpu.sync_copy(d_h.at[pl.ds(base, BPW)], idx_v)
    pltpu.sync_copy(m_h.at[pl.ds(base, BPW)], m_v)

    def issue(b, g, s):
        pltpu.async_copy(tab_h.at[idx_v.at[b, pl.ds(0, SLOTS)]], g, s)

    def wait(b, g, s):
        pltpu.make_async_copy(tab_h.at[idx_v.at[b, pl.ds(0, SLOTS)]], g, s).wait()

    def compute(b, g):
        mrow = [m_v[b, pl.ds(c * 16, 16)] for c in range(2 * D // 16)]
        z = jnp.zeros((16,), jnp.float32)
        accs1 = [z] * NCHUNK
        accs2 = [z] * NCHUNK
        for l in range(L):
            c, j = divmod(l, 16)
            mv1 = jnp.full((16,), mrow[c][j], jnp.float32)
            mv2 = jnp.full((16,), mrow[NCHUNK + c][j], jnp.float32)
            for k in range(NCHUNK):
                accs1[k] = accs1[k] + g[l, pl.ds(k * 16, 16)] * mv1
                accs2[k] = accs2[k] + g[LP + l, pl.ds(k * 16, 16)] * mv2
        for k in range(NCHUNK):
            out_v[b, pl.ds(k * 16, 16)] = accs1[k]
        for k in range(NCHUNK):
            out_v[b, pl.ds(D + k * 16, 16)] = accs2[k]

    for j in range(NBUF):
        issue(j, gs[j], ss[j])

    def step(t, carry):
        for j in range(NBUF):
            b = t * NBUF + j
            wait(b, gs[j], ss[j])
            compute(b, gs[j])

            @pl.when(b + NBUF < BPW)
            def _():
                issue(b + NBUF, gs[j], ss[j])

        return carry

    lax.fori_loop(0, BPW // NBUF, step, 0)

    pltpu.sync_copy(out_v, out_h.at[pl.ds(base, BPW)])


def _pool(d, m, table2):
    mesh = plsc.VectorSubcoreMesh(core_axis_name="c", subcore_axis_name="s")
    f = functools.partial(
        pl.kernel,
        out_type=jax.ShapeDtypeStruct((B, 2 * D), jnp.float32),
        mesh=mesh,
        scratch_types=[
            pltpu.VMEM((BPW, SLOTS), jnp.int32),
            pltpu.VMEM((BPW, 2 * D), jnp.float32),
            pltpu.VMEM((SLOTS, 2 * D), jnp.float32),
            pltpu.VMEM((SLOTS, 2 * D), jnp.float32),
            pltpu.VMEM((SLOTS, 2 * D), jnp.float32),
            pltpu.VMEM((SLOTS, 2 * D), jnp.float32),
            pltpu.VMEM((BPW, 2 * D), jnp.float32),
            pltpu.SemaphoreType.DMA,
            pltpu.SemaphoreType.DMA,
            pltpu.SemaphoreType.DMA,
            pltpu.SemaphoreType.DMA,
        ],
        compiler_params=pltpu.CompilerParams(use_tc_tiling_on_sc=False),
    )(_pool_kernel)
    return f(d, m, table2)


PADC = 2048


def _padtab_body(tt_ref, p_ref, o_ref):
    o_ref[...] = lax.dot_general(
        tt_ref[...], p_ref[...], (((0,), (0,)), ((), ())),
        preferred_element_type=jnp.float32)


def _padtab(table):
    tt = table.T
    p = jnp.pad(jnp.eye(D, dtype=jnp.float32), ((0, 0), (0, D)))
    return pl.pallas_call(
        _padtab_body,
        grid=(W // PADC,),
        in_specs=[
            pl.BlockSpec((D, PADC), lambda i: (0, i)),
            pl.BlockSpec((D, 2 * D), lambda i: (0, 0)),
        ],
        out_specs=pl.BlockSpec((PADC, 2 * D), lambda i: (i, 0)),
        out_shape=jax.ShapeDtypeStruct((W, 2 * D), jnp.float32),
    )(tt, p)


def _mlp_body(opt_ref, w1_ref, b1_ref, w2_ref, b2_ref, o_ref):
    opt = opt_ref[...] * (1.0 / L)
    h = jnp.dot(opt, w1_ref[...], preferred_element_type=jnp.float32)
    h = h + b1_ref[...]
    o = jnp.dot(h, w2_ref[...], preferred_element_type=jnp.float32)
    o_ref[...] = o + b2_ref[...]


def kernel(data_x1, mask_x1, data_x2, mask_x2, table, W1, b1, W2, b2):
    iota_b = jnp.arange(B, dtype=jnp.int32)[:, None]
    iota_j = jnp.arange(LP - L, dtype=jnp.int32)[None, :]
    pad1 = (iota_b * 2 * (LP - L) + iota_j) % W
    pad2 = (iota_b * 2 * (LP - L) + (LP - L) + iota_j) % W
    d1 = jnp.concatenate((data_x1.astype(jnp.int32), pad1), axis=1)
    d2 = jnp.concatenate((data_x2.astype(jnp.int32), pad2), axis=1)
    d = jnp.concatenate((d1, d2), axis=1)
    m1 = jnp.pad(mask_x1, ((0, 0), (0, D - L)))
    m2 = jnp.pad(mask_x2, ((0, 0), (0, D - L)))
    m = jnp.concatenate((m1, m2), axis=1)
    table2 = _padtab(table)
    opt = _pool(d, m, table2)
    return pl.pallas_call(
        _mlp_body,
        out_shape=jax.ShapeDtypeStruct((B, 2), jnp.float32),
    )(opt, W1, b1.reshape(1, -1), W2, b2.reshape(1, -1))
